# baseline (speedup 1.0000x reference)
import functools

import jax
import jax.numpy as jnp
from jax import lax
from jax.experimental import pallas as pl
from jax.experimental.pallas import tpu as pltpu
from jax.experimental.pallas import tpu_sc as plsc

N = 4096
NNZ = 167772

NC = 2
NS = 16
LANES = 16

WS = 10368
ENT = WS // LANES
TW = 1888
TENT = TW // LANES

PASS_ROWS = [176] * 11 + [112]
CHUNK = max(PASS_ROWS) * N
ZW = 32768


def _body(idx_hbm, vals_hbm, trow_hbm, tcol_hbm, tval_hbm, out_hbm,
          ent_v, vals_v, idx_b, tail_r, tail_c, tail_v, tail_i,
          zero_b, acc, sem):
    c = lax.axis_index("c")
    s = lax.axis_index("s")
    shard = s * WS

    pltpu.sync_copy(idx_hbm.at[:, pl.ds(shard, WS)], ent_v)
    pltpu.sync_copy(vals_hbm.at[pl.ds(shard, WS)], vals_v)

    @pl.when(s == 0)
    def _stage_tail():
        pltpu.sync_copy(trow_hbm, tail_r)
        pltpu.sync_copy(tcol_hbm, tail_c)
        pltpu.sync_copy(tval_hbm, tail_v)

    zvec = jnp.zeros((LANES,), jnp.float32)

    def zb_body(i, carry):
        zero_b[pl.ds(i * LANES, LANES)] = zvec
        return carry

    lax.fori_loop(0, ZW // LANES, zb_body, 0)

    row_off = 0
    for rows_p in PASS_ROWS:
        base = c * (N // NC) + row_off
        row_off += rows_p
        ts = rows_p * N // NS

        zdone = 0
        while zdone < ts:
            zn = min(ZW, ts - zdone)
            pltpu.sync_copy(zero_b.at[pl.ds(0, zn)],
                            acc.at[pl.ds(s * ts + zdone, zn)])
            zdone += zn
        plsc.subcore_barrier()

        def scan_body(i, carry):
            off = i * LANES
            row = ent_v[0, pl.ds(off, LANES)]
            col = ent_v[1, pl.ds(off, LANES)]
            rel = row - base
            m = (rel >= 0) & (rel < rows_p)
            idx_b[pl.ds(off, LANES)] = jnp.where(m, (rel << 12) + col, -1)
            return carry

        lax.fori_loop(0, ENT, scan_body, 0)

        pltpu.sync_copy(vals_v,
                        acc.at[plsc.Indices(idx_b, ignored_value=-1)],
                        add=True)

        @pl.when(s == 0)
        def _tail_pass():
            def tail_body(i, carry):
                off = i * LANES
                row = tail_r[pl.ds(off, LANES)]
                col = tail_c[pl.ds(off, LANES)]
                rel = row - base
                m = (rel >= 0) & (rel < rows_p)
                tail_i[pl.ds(off, LANES)] = jnp.where(m, (rel << 12) + col, -1)
                return carry

            lax.fori_loop(0, TENT, tail_body, 0)
            pltpu.sync_copy(tail_v,
                            acc.at[plsc.Indices(tail_i, ignored_value=-1)],
                            add=True)

        plsc.subcore_barrier()

        nr = rows_p // NS
        cps = [pltpu.async_copy(acc.at[pl.ds((s * nr + r) * N, N)],
                                out_hbm.at[base + s * nr + r, :], sem)
               for r in range(nr)]
        for cp in cps:
            cp.wait()
        plsc.subcore_barrier()


_mesh = plsc.VectorSubcoreMesh(core_axis_name="c", subcore_axis_name="s",
                               num_cores=NC, num_subcores=NS)

_sc_call = functools.partial(
    pl.kernel,
    out_type=jax.ShapeDtypeStruct((N, N), jnp.float32),
    mesh=_mesh,
    scratch_types=[
        pltpu.VMEM((2, WS), jnp.int32),
        pltpu.VMEM((WS,), jnp.float32),
        pltpu.VMEM((WS,), jnp.int32),
        pltpu.VMEM((TW,), jnp.int32),
        pltpu.VMEM((TW,), jnp.int32),
        pltpu.VMEM((TW,), jnp.float32),
        pltpu.VMEM((TW,), jnp.int32),
        pltpu.VMEM((ZW,), jnp.float32),
        pltpu.VMEM_SHARED((CHUNK,), jnp.float32),
        pltpu.SemaphoreType.DMA,
    ],
)(_body)


def kernel(indices, values):
    idx = indices.astype(jnp.int32)
    vals = values.astype(jnp.float32)
    nt = NNZ - NS * WS
    zpad = jnp.zeros((TW - nt,), jnp.int32)
    trow = jnp.concatenate([idx[0, NS * WS:], zpad])
    tcol = jnp.concatenate([idx[1, NS * WS:], zpad])
    tval = jnp.concatenate([vals[NS * WS:], jnp.zeros((TW - nt,), jnp.float32)])
    return _sc_call(idx, vals, trow, tcol, tval)


# speedup vs baseline: 1.0635x; 1.0635x over previous
import functools

import jax
import jax.numpy as jnp
from jax import lax
from jax.experimental import pallas as pl
from jax.experimental.pallas import tpu as pltpu
from jax.experimental.pallas import tpu_sc as plsc

N = 4096
NNZ = 167772

NC = 2
NS = 16
LANES = 16

WS = 10368
ENT = WS // LANES
TB = 128
TW = NS * TB
TBENT = TB // LANES

PASS_ROWS = [176] * 11 + [112]
CHUNK = max(PASS_ROWS) * N
ZW = 32768


def _body(idx_hbm, vals_hbm, trow_hbm, tcol_hbm, tval_hbm, out_hbm,
          ent_v, vals_v, idx_b, tail_r, tail_c,
          zero_b, acc, sem):
    c = lax.axis_index("c")
    s = lax.axis_index("s")
    shard = s * WS

    pltpu.sync_copy(idx_hbm.at[:, pl.ds(shard, WS)], ent_v)
    pltpu.sync_copy(vals_hbm.at[pl.ds(shard, WS)], vals_v.at[pl.ds(0, WS)])
    pltpu.sync_copy(trow_hbm.at[pl.ds(s * TB, TB)], tail_r)
    pltpu.sync_copy(tcol_hbm.at[pl.ds(s * TB, TB)], tail_c)
    pltpu.sync_copy(tval_hbm.at[pl.ds(s * TB, TB)], vals_v.at[pl.ds(WS, TB)])

    zvec = jnp.zeros((LANES,), jnp.float32)

    def zb_body(i, carry):
        zero_b[pl.ds(i * LANES, LANES)] = zvec
        return carry

    lax.fori_loop(0, ZW // LANES, zb_body, 0)

    row_off = 0
    for rows_p in PASS_ROWS:
        base = c * (N // NC) + row_off
        row_off += rows_p
        ts = rows_p * N // NS

        zdone = 0
        while zdone < ts:
            zn = min(ZW, ts - zdone)
            pltpu.sync_copy(zero_b.at[pl.ds(0, zn)],
                            acc.at[pl.ds(s * ts + zdone, zn)])
            zdone += zn
        plsc.subcore_barrier()

        def scan_body(i, carry):
            off = i * LANES
            row = ent_v[0, pl.ds(off, LANES)]
            col = ent_v[1, pl.ds(off, LANES)]
            rel = row - base
            m = (rel >= 0) & (rel < rows_p)
            idx_b[pl.ds(off, LANES)] = jnp.where(m, (rel << 12) + col, -1)
            return carry

        lax.fori_loop(0, ENT, scan_body, 0)

        def tail_body(i, carry):
            off = i * LANES
            row = tail_r[pl.ds(off, LANES)]
            col = tail_c[pl.ds(off, LANES)]
            rel = row - base
            m = (rel >= 0) & (rel < rows_p)
            idx_b[pl.ds(WS + off, LANES)] = jnp.where(m, (rel << 12) + col, -1)
            return carry

        lax.fori_loop(0, TBENT, tail_body, 0)

        pltpu.sync_copy(vals_v,
                        acc.at[plsc.Indices(idx_b, ignored_value=-1)],
                        add=True)
        plsc.subcore_barrier()

        nr = rows_p // NS
        cps = [pltpu.async_copy(acc.at[pl.ds((s * nr + r) * N, N)],
                                out_hbm.at[base + s * nr + r, :], sem)
               for r in range(nr)]
        for cp in cps:
            cp.wait()
        plsc.subcore_barrier()


_mesh = plsc.VectorSubcoreMesh(core_axis_name="c", subcore_axis_name="s",
                               num_cores=NC, num_subcores=NS)

_sc_call = functools.partial(
    pl.kernel,
    out_type=jax.ShapeDtypeStruct((N, N), jnp.float32),
    mesh=_mesh,
    scratch_types=[
        pltpu.VMEM((2, WS), jnp.int32),
        pltpu.VMEM((WS + TB,), jnp.float32),
        pltpu.VMEM((WS + TB,), jnp.int32),
        pltpu.VMEM((TB,), jnp.int32),
        pltpu.VMEM((TB,), jnp.int32),
        pltpu.VMEM((ZW,), jnp.float32),
        pltpu.VMEM_SHARED((CHUNK,), jnp.float32),
        pltpu.SemaphoreType.DMA,
    ],
)(_body)


def kernel(indices, values):
    idx = indices.astype(jnp.int32)
    vals = values.astype(jnp.float32)
    nt = NNZ - NS * WS
    zpad = jnp.zeros((TW - nt,), jnp.int32)
    trow = jnp.concatenate([idx[0, NS * WS:], zpad])
    tcol = jnp.concatenate([idx[1, NS * WS:], zpad])
    tval = jnp.concatenate([vals[NS * WS:], jnp.zeros((TW - nt,), jnp.float32)])
    return _sc_call(idx, vals, trow, tcol, tval)


# speedup vs baseline: 1.1290x; 1.0616x over previous
import functools

import jax
import jax.numpy as jnp
from jax import lax
from jax.experimental import pallas as pl
from jax.experimental.pallas import tpu as pltpu
from jax.experimental.pallas import tpu_sc as plsc

N = 4096
NNZ = 167772

NC = 2
NS = 16
LANES = 16

WS = 10368
ENT = WS // LANES
TB = 128
TW = NS * TB
TBENT = TB // LANES

PASS_ROWS = [176] * 11 + [112]
CHUNK = max(PASS_ROWS) * N
ZW = 32768


def _body(rows_hbm, cols_hbm, vals_hbm, trow_hbm, tcol_hbm, tval_hbm,
          out_hbm, rows_v, cols_v, vals_v, idx_b, tail_r, tail_c,
          zero_b, acc, sem):
    c = lax.axis_index("c")
    s = lax.axis_index("s")
    shard = s * WS

    pltpu.sync_copy(rows_hbm.at[pl.ds(shard, WS)], rows_v)
    pltpu.sync_copy(cols_hbm.at[pl.ds(shard, WS)], cols_v)
    pltpu.sync_copy(vals_hbm.at[pl.ds(shard, WS)], vals_v.at[pl.ds(0, WS)])
    pltpu.sync_copy(trow_hbm.at[pl.ds(s * TB, TB)], tail_r)
    pltpu.sync_copy(tcol_hbm.at[pl.ds(s * TB, TB)], tail_c)
    pltpu.sync_copy(tval_hbm.at[pl.ds(s * TB, TB)], vals_v.at[pl.ds(WS, TB)])

    zvec = jnp.zeros((LANES,), jnp.float32)

    def zb_body(i, carry):
        zero_b[pl.ds(i * LANES, LANES)] = zvec
        return carry

    lax.fori_loop(0, ZW // LANES, zb_body, 0)

    row_off = 0
    for rows_p in PASS_ROWS:
        base = c * (N // NC) + row_off
        row_off += rows_p
        ts = rows_p * N // NS

        zdone = 0
        while zdone < ts:
            zn = min(ZW, ts - zdone)
            pltpu.sync_copy(zero_b.at[pl.ds(0, zn)],
                            acc.at[pl.ds(s * ts + zdone, zn)])
            zdone += zn
        plsc.subcore_barrier()

        def scan_body(i, carry):
            off = i * LANES
            row = rows_v[pl.ds(off, LANES)]
            col = cols_v[pl.ds(off, LANES)]
            rel = row - base
            m = (rel >= 0) & (rel < rows_p)
            idx_b[pl.ds(off, LANES)] = jnp.where(m, (rel << 12) + col, -1)
            return carry

        lax.fori_loop(0, ENT, scan_body, 0)

        def tail_body(i, carry):
            off = i * LANES
            row = tail_r[pl.ds(off, LANES)]
            col = tail_c[pl.ds(off, LANES)]
            rel = row - base
            m = (rel >= 0) & (rel < rows_p)
            idx_b[pl.ds(WS + off, LANES)] = jnp.where(m, (rel << 12) + col, -1)
            return carry

        lax.fori_loop(0, TBENT, tail_body, 0)

        pltpu.sync_copy(vals_v,
                        acc.at[plsc.Indices(idx_b, ignored_value=-1)],
                        add=True)
        plsc.subcore_barrier()

        nr = rows_p // NS
        cps = [pltpu.async_copy(acc.at[pl.ds((s * nr + r) * N, N)],
                                out_hbm.at[base + s * nr + r, :], sem)
               for r in range(nr)]
        for cp in cps:
            cp.wait()
        plsc.subcore_barrier()


_mesh = plsc.VectorSubcoreMesh(core_axis_name="c", subcore_axis_name="s",
                               num_cores=NC, num_subcores=NS)

_sc_call = functools.partial(
    pl.kernel,
    out_type=jax.ShapeDtypeStruct((N, N), jnp.float32),
    mesh=_mesh,
    scratch_types=[
        pltpu.VMEM((WS,), jnp.int32),
        pltpu.VMEM((WS,), jnp.int32),
        pltpu.VMEM((WS + TB,), jnp.float32),
        pltpu.VMEM((WS + TB,), jnp.int32),
        pltpu.VMEM((TB,), jnp.int32),
        pltpu.VMEM((TB,), jnp.int32),
        pltpu.VMEM((ZW,), jnp.float32),
        pltpu.VMEM_SHARED((CHUNK,), jnp.float32),
        pltpu.SemaphoreType.DMA,
    ],
)(_body)


def kernel(indices, values):
    idx = indices.astype(jnp.int32)
    vals = values.astype(jnp.float32)
    rows = idx[0]
    cols = idx[1]
    nt = NNZ - NS * WS
    zpad = jnp.zeros((TW - nt,), jnp.int32)
    trow = jnp.concatenate([rows[NS * WS:], zpad])
    tcol = jnp.concatenate([cols[NS * WS:], zpad])
    tval = jnp.concatenate([vals[NS * WS:], jnp.zeros((TW - nt,), jnp.float32)])
    return _sc_call(rows, cols, vals, trow, tcol, tval)


# speedup vs baseline: 1.3368x; 1.1840x over previous
import functools

import jax
import jax.numpy as jnp
from jax import lax
from jax.experimental import pallas as pl
from jax.experimental.pallas import tpu as pltpu
from jax.experimental.pallas import tpu_sc as plsc

N = 4096
NNZ = 167772

NC = 2
NS = 16
LANES = 16

WS = 10368
ENT = WS // LANES
TB = 128
TW = NS * TB
TBENT = TB // LANES

PASS_ROWS = [176] * 11 + [112]
CHUNK = max(PASS_ROWS) * N
ZW = 32768


def _body(rows_hbm, cols_hbm, vals_hbm, trow_hbm, tcol_hbm, tval_hbm,
          out_hbm, rows_v, cols_v, vals_v, idx_b, tail_r, tail_c,
          zero_b, acc, sem, zsem):
    c = lax.axis_index("c")
    s = lax.axis_index("s")
    shard = s * WS

    pltpu.sync_copy(rows_hbm.at[pl.ds(shard, WS)], rows_v)
    pltpu.sync_copy(cols_hbm.at[pl.ds(shard, WS)], cols_v)
    pltpu.sync_copy(vals_hbm.at[pl.ds(shard, WS)], vals_v.at[pl.ds(0, WS)])
    pltpu.sync_copy(trow_hbm.at[pl.ds(s * TB, TB)], tail_r)
    pltpu.sync_copy(tcol_hbm.at[pl.ds(s * TB, TB)], tail_c)
    pltpu.sync_copy(tval_hbm.at[pl.ds(s * TB, TB)], vals_v.at[pl.ds(WS, TB)])

    zvec = jnp.zeros((LANES,), jnp.float32)

    def zb_body(i, carry):
        zero_b[pl.ds(i * LANES, LANES)] = zvec
        return carry

    lax.fori_loop(0, ZW // LANES, zb_body, 0)

    pend_out = []
    row_off = 0
    for rows_p in PASS_ROWS:
        base = c * (N // NC) + row_off
        row_off += rows_p
        ts = rows_p * N // NS
        nr = rows_p // NS

        def scan_body(i, carry):
            off = i * LANES
            row = rows_v[pl.ds(off, LANES)]
            col = cols_v[pl.ds(off, LANES)]
            rel = row - base
            m = (rel >= 0) & (rel < rows_p)
            idx_b[pl.ds(off, LANES)] = jnp.where(m, (rel << 12) + col, -1)
            return carry

        lax.fori_loop(0, ENT // 2, scan_body, 0)

        for cp in pend_out:
            cp.wait()
        plsc.subcore_barrier()

        zcps = []
        zdone = 0
        while zdone < ts:
            zn = min(ZW, ts - zdone)
            zcps.append(pltpu.async_copy(zero_b.at[pl.ds(0, zn)],
                                         acc.at[pl.ds(s * ts + zdone, zn)],
                                         zsem))
            zdone += zn

        lax.fori_loop(ENT // 2, ENT, scan_body, 0)

        def tail_body(i, carry):
            off = i * LANES
            row = tail_r[pl.ds(off, LANES)]
            col = tail_c[pl.ds(off, LANES)]
            rel = row - base
            m = (rel >= 0) & (rel < rows_p)
            idx_b[pl.ds(WS + off, LANES)] = jnp.where(m, (rel << 12) + col, -1)
            return carry

        lax.fori_loop(0, TBENT, tail_body, 0)

        for cp in zcps:
            cp.wait()
        plsc.subcore_barrier()

        pltpu.sync_copy(vals_v,
                        acc.at[plsc.Indices(idx_b, ignored_value=-1)],
                        add=True)
        plsc.subcore_barrier()

        pend_out = [pltpu.async_copy(acc.at[pl.ds((s * nr + r) * N, N)],
                                     out_hbm.at[base + s * nr + r, :], sem)
                    for r in range(nr)]

    for cp in pend_out:
        cp.wait()


_mesh = plsc.VectorSubcoreMesh(core_axis_name="c", subcore_axis_name="s",
                               num_cores=NC, num_subcores=NS)

_sc_call = functools.partial(
    pl.kernel,
    out_type=jax.ShapeDtypeStruct((N, N), jnp.float32),
    mesh=_mesh,
    scratch_types=[
        pltpu.VMEM((WS,), jnp.int32),
        pltpu.VMEM((WS,), jnp.int32),
        pltpu.VMEM((WS + TB,), jnp.float32),
        pltpu.VMEM((WS + TB,), jnp.int32),
        pltpu.VMEM((TB,), jnp.int32),
        pltpu.VMEM((TB,), jnp.int32),
        pltpu.VMEM((ZW,), jnp.float32),
        pltpu.VMEM_SHARED((CHUNK,), jnp.float32),
        pltpu.SemaphoreType.DMA,
        pltpu.SemaphoreType.DMA,
    ],
)(_body)


def kernel(indices, values):
    idx = indices.astype(jnp.int32)
    vals = values.astype(jnp.float32)
    rows = idx[0]
    cols = idx[1]
    nt = NNZ - NS * WS
    zpad = jnp.zeros((TW - nt,), jnp.int32)
    trow = jnp.concatenate([rows[NS * WS:], zpad])
    tcol = jnp.concatenate([cols[NS * WS:], zpad])
    tval = jnp.concatenate([vals[NS * WS:], jnp.zeros((TW - nt,), jnp.float32)])
    return _sc_call(rows, cols, vals, trow, tcol, tval)
